# baseline (speedup 1.0000x reference)
import jax
import jax.numpy as jnp
from jax import lax
from jax.experimental import pallas as pl
from jax.experimental.pallas import tpu as pltpu
from jax.experimental.pallas import tpu_sc as plsc

B = 64
L = 512
D = 1024
EPS = 1e-12
LANES = 16
NW = 32
PW = L // NW
TB = 64
NT = B * L // TB


def _sc_gather_body(idst_h, tok_h, out_h, idxt_v, idx_v, rows,
                    gsem0, gsem1, osem0, osem1):
    wid = lax.axis_index("s") * 2 + lax.axis_index("c")
    p0 = wid * PW
    gsem = (gsem0, gsem1)
    osem = (osem0, osem1)

    pltpu.sync_copy(idst_h.at[pl.ds(p0, PW), :], idxt_v)
    lane = lax.iota(jnp.int32, LANES)

    def _tr(b, _):
        idx_v[b, :] = plsc.load_gather(
            idxt_v, [lane, jnp.full((LANES,), b, jnp.int32)])
        return 0
    lax.fori_loop(0, B, _tr, 0)

    def _gather(b, par):
        return pltpu.async_copy(tok_h.at[idx_v.at[b]], rows.at[par], gsem[par])

    _gather(0, 0)

    def bstep(b2, _):
        for par in (0, 1):
            b = 2 * b2 + par

            @pl.when(b >= 1)
            def _():
                pltpu.make_async_copy(rows.at[1 - par],
                                      out_h.at[b - 1, pl.ds(p0, PW), :],
                                      osem[1 - par]).wait()

            @pl.when(b + 1 < B)
            def _():
                _gather(b + 1, 1 - par)

            pltpu.make_async_copy(tok_h.at[idx_v.at[b]], rows.at[par],
                                  gsem[par]).wait()
            pltpu.async_copy(rows.at[par], out_h.at[b, pl.ds(p0, PW), :],
                             osem[par])
        return 0

    lax.fori_loop(0, B // 2, bstep, 0)
    pltpu.make_async_copy(rows.at[1], out_h.at[B - 1, pl.ds(p0, PW), :],
                          osem[1]).wait()


def _tc_ln_body(x_ref, pos_ref, ttf_ref, seg_ref, gam_ref, bet_ref, o_ref):
    x = x_ref[...] + pos_ref[...]
    seg0 = seg_ref[0:1, :]
    segd = seg_ref[1:2, :] - seg0
    x = x + seg0 + ttf_ref[...] * segd
    mean = jnp.mean(x, axis=-1, keepdims=True)
    xc = x - mean
    var = jnp.mean(xc * xc, axis=-1, keepdims=True)
    y = xc * lax.rsqrt(var + EPS)
    o_ref[...] = y * gam_ref[...] + bet_ref[...]


@jax.jit
def kernel(input_ids, token_type_ids, tok_emb, pos_emb, seg_emb, gamma, beta):
    mesh = plsc.VectorSubcoreMesh(core_axis_name="c", subcore_axis_name="s",
                                  num_cores=2, num_subcores=16)
    gathered = pl.kernel(
        _sc_gather_body,
        out_type=jax.ShapeDtypeStruct((B, L, D), jnp.float32),
        mesh=mesh,
        compiler_params=pltpu.CompilerParams(needs_layout_passes=False),
        scratch_types=[
            pltpu.VMEM((PW, B), jnp.int32),
            pltpu.VMEM((B, PW), jnp.int32),
            pltpu.VMEM((2, PW, D), jnp.float32),
            pltpu.SemaphoreType.DMA,
            pltpu.SemaphoreType.DMA,
            pltpu.SemaphoreType.DMA,
            pltpu.SemaphoreType.DMA,
        ],
    )(input_ids.T, tok_emb)

    ttf = token_type_ids.reshape(B * L, 1).astype(jnp.float32)
    out = pl.pallas_call(
        _tc_ln_body,
        grid=(NT,),
        in_specs=[
            pl.BlockSpec((TB, D), lambda i: (i, 0)),
            pl.BlockSpec((TB, D), lambda i: (i % (L // TB), 0)),
            pl.BlockSpec((TB, 1), lambda i: (i, 0)),
            pl.BlockSpec((2, D), lambda i: (0, 0)),
            pl.BlockSpec((1, D), lambda i: (0, 0)),
            pl.BlockSpec((1, D), lambda i: (0, 0)),
        ],
        out_specs=pl.BlockSpec((TB, D), lambda i: (i, 0)),
        out_shape=jax.ShapeDtypeStruct((B * L, D), jnp.float32),
        compiler_params=pltpu.CompilerParams(
            dimension_semantics=("arbitrary",)),
    )(gathered.reshape(B * L, D), pos_emb, ttf, seg_emb,
      gamma.reshape(1, D), beta.reshape(1, D))
    return out.reshape(B, L, D)


# speedup vs baseline: 1.7652x; 1.7652x over previous
import jax
import jax.numpy as jnp
from jax import lax
from jax.experimental import pallas as pl
from jax.experimental.pallas import tpu as pltpu
from jax.experimental.pallas import tpu_sc as plsc

B = 64
L = 512
D = 1024
EPS = 1e-12
LANES = 16
NW = 32
PW = L // NW
TB = 256
NT = B * L // TB


def _sc_gather_body(idst_h, tok_h, out_h, idxt_v, idx_v, rows,
                    gsem0, gsem1, osem0, osem1):
    wid = lax.axis_index("s") * 2 + lax.axis_index("c")
    p0 = wid * PW
    gsem = (gsem0, gsem1)
    osem = (osem0, osem1)

    pltpu.sync_copy(idst_h.at[pl.ds(p0, PW), :], idxt_v)
    lane = lax.iota(jnp.int32, LANES)

    def _tr(b, _):
        idx_v[b, :] = plsc.load_gather(
            idxt_v, [lane, jnp.full((LANES,), b, jnp.int32)])
        return 0
    lax.fori_loop(0, B, _tr, 0)

    def _gather(b, par):
        return pltpu.async_copy(tok_h.at[idx_v.at[b]], rows.at[par], gsem[par])

    _gather(0, 0)

    def bstep(b2, _):
        for par in (0, 1):
            b = 2 * b2 + par

            @pl.when(b >= 1)
            def _():
                pltpu.make_async_copy(rows.at[1 - par],
                                      out_h.at[b - 1, pl.ds(p0, PW), :],
                                      osem[1 - par]).wait()

            @pl.when(b + 1 < B)
            def _():
                _gather(b + 1, 1 - par)

            pltpu.make_async_copy(tok_h.at[idx_v.at[b]], rows.at[par],
                                  gsem[par]).wait()
            pltpu.async_copy(rows.at[par], out_h.at[b, pl.ds(p0, PW), :],
                             osem[par])
        return 0

    lax.fori_loop(0, B // 2, bstep, 0)
    pltpu.make_async_copy(rows.at[1], out_h.at[B - 1, pl.ds(p0, PW), :],
                          osem[1]).wait()


def _tc_ln_body(x_ref, pos_ref, ttf_ref, seg_ref, gam_ref, bet_ref, o_ref):
    x = x_ref[...] + pos_ref[...]
    seg0 = seg_ref[0:1, :]
    segd = seg_ref[1:2, :] - seg0
    x = x + seg0 + ttf_ref[...] * segd
    mean = jnp.mean(x, axis=-1, keepdims=True)
    xc = x - mean
    var = jnp.mean(xc * xc, axis=-1, keepdims=True)
    y = xc * lax.rsqrt(var + EPS)
    o_ref[...] = y * gam_ref[...] + bet_ref[...]


@jax.jit
def kernel(input_ids, token_type_ids, tok_emb, pos_emb, seg_emb, gamma, beta):
    mesh = plsc.VectorSubcoreMesh(core_axis_name="c", subcore_axis_name="s",
                                  num_cores=2, num_subcores=16)
    gathered = pl.kernel(
        _sc_gather_body,
        out_type=jax.ShapeDtypeStruct((B, L, D), jnp.float32),
        mesh=mesh,
        compiler_params=pltpu.CompilerParams(needs_layout_passes=False),
        scratch_types=[
            pltpu.VMEM((PW, B), jnp.int32),
            pltpu.VMEM((B, PW), jnp.int32),
            pltpu.VMEM((2, PW, D), jnp.float32),
            pltpu.SemaphoreType.DMA,
            pltpu.SemaphoreType.DMA,
            pltpu.SemaphoreType.DMA,
            pltpu.SemaphoreType.DMA,
        ],
    )(input_ids.T, tok_emb)

    ttf = token_type_ids.reshape(B * L, 1).astype(jnp.float32)
    out = pl.pallas_call(
        _tc_ln_body,
        grid=(NT,),
        in_specs=[
            pl.BlockSpec((TB, D), lambda i: (i, 0)),
            pl.BlockSpec((TB, D), lambda i: (i % (L // TB), 0)),
            pl.BlockSpec((TB, 1), lambda i: (i, 0)),
            pl.BlockSpec((2, D), lambda i: (0, 0)),
            pl.BlockSpec((1, D), lambda i: (0, 0)),
            pl.BlockSpec((1, D), lambda i: (0, 0)),
        ],
        out_specs=pl.BlockSpec((TB, D), lambda i: (i, 0)),
        out_shape=jax.ShapeDtypeStruct((B * L, D), jnp.float32),
        compiler_params=pltpu.CompilerParams(
            dimension_semantics=("parallel",)),
    )(gathered.reshape(B * L, D), pos_emb, ttf, seg_emb,
      gamma.reshape(1, D), beta.reshape(1, D))
    return out.reshape(B, L, D)
